# Initial kernel scaffold; baseline (speedup 1.0000x reference)
import jax
import jax.numpy as jnp
from jax.experimental import pallas as pl


def kernel(x, pos_ids, weight):
    raise NotImplementedError("write your pallas kernel here")


# speedup vs baseline: 1.1547x; 1.1547x over previous
import functools

import jax
import jax.numpy as jnp
from jax import lax
from jax.experimental import pallas as pl
from jax.experimental.pallas import tpu as pltpu
from jax.experimental.pallas import tpu_sc as plsc

_LANES = 16


@functools.lru_cache(maxsize=None)
def _build(N: int, D: int, V: int):
    info = plsc.get_sparse_core_info()
    NC, NS = info.num_cores, info.num_subcores
    NW = NC * NS

    assert N % NW == 0 and D % _LANES == 0
    rows_per_w = N // NW
    C = 32
    assert rows_per_w % C == 0
    n_chunks = rows_per_w // C

    mesh = plsc.VectorSubcoreMesh(core_axis_name="c", subcore_axis_name="s")

    @functools.partial(
        pl.kernel,
        mesh=mesh,
        out_type=jax.ShapeDtypeStruct((N, D), jnp.float32),
        scratch_types=[
            pltpu.VMEM((C,), jnp.int32),
            pltpu.VMEM((C, D), jnp.float32),
            pltpu.VMEM((C, D), jnp.float32),
            pltpu.SemaphoreType.DMA,
        ],
    )
    def k(x_hbm, idx_hbm, w_hbm, out_hbm, idx_v, g_v, x_v, sem):
        wid = lax.axis_index("s") * NC + lax.axis_index("c")
        base = wid * rows_per_w

        def chunk_body(cidx, carry):
            row0 = base + cidx * C
            pltpu.sync_copy(idx_hbm.at[pl.ds(row0, C)], idx_v)
            gcp = pltpu.async_copy(w_hbm.at[idx_v], g_v, sem)
            pltpu.sync_copy(x_hbm.at[pl.ds(row0, C), :], x_v)
            gcp.wait()

            def row_body(r, c2):
                for j in range(D // _LANES):
                    sl = pl.ds(j * _LANES, _LANES)
                    plsc.addupdate(x_v.at[r, sl], g_v[r, sl])
                return c2

            lax.fori_loop(0, C, row_body, 0)
            pltpu.sync_copy(x_v, out_hbm.at[pl.ds(row0, C), :])
            return carry

        lax.fori_loop(0, n_chunks, chunk_body, 0)

    return k


def kernel(x, pos_ids, weight):
    B, L, D = x.shape
    V = weight.shape[0]
    N = B * L
    x_flat = x.reshape(N, D)
    idx_flat = pos_ids.reshape(N).astype(jnp.int32)
    out = _build(N, D, V)(x_flat, idx_flat, weight)
    return out.reshape(B, L, D)


# speedup vs baseline: 1.8072x; 1.5651x over previous
import functools

import jax
import jax.numpy as jnp
from jax import lax
from jax.experimental import pallas as pl
from jax.experimental.pallas import tpu as pltpu
from jax.experimental.pallas import tpu_sc as plsc

_LANES = 16


@functools.lru_cache(maxsize=None)
def _build(N: int, D: int, V: int):
    info = plsc.get_sparse_core_info()
    NC, NS = info.num_cores, info.num_subcores
    NW = NC * NS

    assert N % NW == 0 and D % _LANES == 0
    rows_per_w = N // NW
    C = 16
    assert rows_per_w % C == 0
    n_chunks = rows_per_w // C
    assert n_chunks % 4 == 0 and n_chunks >= 8

    mesh = plsc.VectorSubcoreMesh(core_axis_name="c", subcore_axis_name="s")

    @functools.partial(
        pl.kernel,
        mesh=mesh,
        out_type=jax.ShapeDtypeStruct((N, D), jnp.float32),
        scratch_types=[
            pltpu.VMEM((rows_per_w,), jnp.int32),
            pltpu.VMEM((4, C, D), jnp.float32),
            pltpu.VMEM((2, C, D), jnp.float32),
            pltpu.SemaphoreType.DMA,
            pltpu.SemaphoreType.DMA,
            pltpu.SemaphoreType.DMA,
            pltpu.SemaphoreType.DMA,
            pltpu.SemaphoreType.DMA,
            pltpu.SemaphoreType.DMA,
            pltpu.SemaphoreType.DMA,
            pltpu.SemaphoreType.DMA,
            pltpu.SemaphoreType.DMA,
            pltpu.SemaphoreType.DMA,
        ],
    )
    def k(x_hbm, idx_hbm, w_hbm, out_hbm, idx_v, g_v, x_v,
          gs0, gs1, gs2, gs3, xs0, xs1, os0, os1, os2, os3):
        wid = lax.axis_index("s") * NC + lax.axis_index("c")
        base = wid * rows_per_w
        gsem = (gs0, gs1, gs2, gs3)
        xsem = (xs0, xs1)
        osem = (os0, os1, os2, os3)

        pltpu.sync_copy(idx_hbm.at[pl.ds(base, rows_per_w)], idx_v)

        def issue_in(c, bg, bx):
            pltpu.async_copy(
                w_hbm.at[idx_v.at[pl.ds(c * C, C)]], g_v.at[bg], gsem[bg])
            pltpu.async_copy(
                x_hbm.at[pl.ds(base + c * C, C), :], x_v.at[bx], xsem[bx])

        def wait_out(bg):
            pltpu.make_async_copy(g_v.at[bg], out_hbm.at[pl.ds(base, C), :],
                                  osem[bg]).wait()

        def step(c, k_):
            bg, bx = k_ % 4, k_ % 2
            nbg, nbx = (k_ + 1) % 4, (k_ + 1) % 2

            @pl.when(c >= 3)
            def _():
                wait_out(nbg)

            @pl.when(c + 1 < n_chunks)
            def _():
                issue_in(c + 1, nbg, nbx)

            pltpu.make_async_copy(w_hbm.at[idx_v.at[pl.ds(0, C)]],
                                  g_v.at[bg], gsem[bg]).wait()
            pltpu.make_async_copy(x_hbm.at[pl.ds(base, C), :],
                                  x_v.at[bx], xsem[bx]).wait()

            def row_body(r, carry):
                for j in range(D // _LANES):
                    sl = pl.ds(j * _LANES, _LANES)
                    plsc.addupdate(g_v.at[bg, r, sl], x_v[bx, r, sl])
                return carry

            lax.fori_loop(0, C, row_body, 0)
            pltpu.async_copy(g_v.at[bg],
                             out_hbm.at[pl.ds(base + c * C, C), :], osem[bg])

        issue_in(0, 0, 0)

        def group(i, carry):
            for k_ in range(4):
                step(4 * i + k_, k_)
            return carry

        lax.fori_loop(0, n_chunks // 4, group, 0)
        for c in (n_chunks - 3, n_chunks - 2, n_chunks - 1):
            wait_out(c % 4)

    return k


def kernel(x, pos_ids, weight):
    B, L, D = x.shape
    V = weight.shape[0]
    N = B * L
    x_flat = x.reshape(N, D)
    idx_flat = pos_ids.reshape(N).astype(jnp.int32)
    out = _build(N, D, V)(x_flat, idx_flat, weight)
    return out.reshape(B, L, D)


# speedup vs baseline: 1.9108x; 1.0573x over previous
import functools

import jax
import jax.numpy as jnp
from jax import lax
from jax.experimental import pallas as pl
from jax.experimental.pallas import tpu as pltpu
from jax.experimental.pallas import tpu_sc as plsc

_LANES = 16


@functools.lru_cache(maxsize=None)
def _build(N: int, D: int, V: int):
    info = plsc.get_sparse_core_info()
    NC, NS = info.num_cores, info.num_subcores
    NW = NC * NS

    assert N % NW == 0 and D % _LANES == 0
    rows_per_w = N // NW
    C = 16
    assert rows_per_w % C == 0
    n_chunks = rows_per_w // C
    assert n_chunks % 4 == 0 and n_chunks >= 8

    mesh = plsc.VectorSubcoreMesh(core_axis_name="c", subcore_axis_name="s")

    @functools.partial(
        pl.kernel,
        mesh=mesh,
        out_type=jax.ShapeDtypeStruct((N, D), jnp.float32),
        scratch_types=[
            pltpu.VMEM((rows_per_w,), jnp.int32),
            pltpu.VMEM((4, C, D), jnp.float32),
            pltpu.VMEM((2, C, D), jnp.float32),
            pltpu.SemaphoreType.DMA,
            pltpu.SemaphoreType.DMA,
            pltpu.SemaphoreType.DMA,
            pltpu.SemaphoreType.DMA,
            pltpu.SemaphoreType.DMA,
            pltpu.SemaphoreType.DMA,
            pltpu.SemaphoreType.DMA,
            pltpu.SemaphoreType.DMA,
            pltpu.SemaphoreType.DMA,
            pltpu.SemaphoreType.DMA,
        ],
    )
    def k(x_hbm, idx_hbm, w_hbm, out_hbm, idx_v, g_v, x_v,
          gs0, gs1, gs2, gs3, xs0, xs1, os0, os1, os2, os3):
        wid = lax.axis_index("s") * NC + lax.axis_index("c")
        base = wid * rows_per_w
        gsem = (gs0, gs1, gs2, gs3)
        xsem = (xs0, xs1)
        osem = (os0, os1, os2, os3)

        pltpu.sync_copy(idx_hbm.at[pl.ds(base, rows_per_w)], idx_v)

        def issue_in(c, bg, bx):
            pltpu.async_copy(
                w_hbm.at[idx_v.at[pl.ds(c * C, C)]], g_v.at[bg], gsem[bg])
            pltpu.async_copy(
                x_hbm.at[pl.ds(base + c * C, C), :], x_v.at[bx], xsem[bx])

        def wait_out(bg):
            pltpu.make_async_copy(g_v.at[bg], out_hbm.at[pl.ds(base, C), :],
                                  osem[bg]).wait()

        def step(c, k_):
            bg, bx = k_ % 4, k_ % 2
            nbg, nbx = (k_ + 1) % 4, (k_ + 1) % 2

            @pl.when(c >= 3)
            def _():
                wait_out(nbg)

            @pl.when(c + 1 < n_chunks)
            def _():
                issue_in(c + 1, nbg, nbx)

            pltpu.make_async_copy(w_hbm.at[idx_v.at[pl.ds(0, C)]],
                                  g_v.at[bg], gsem[bg]).wait()
            pltpu.make_async_copy(x_hbm.at[pl.ds(base, C), :],
                                  x_v.at[bx], xsem[bx]).wait()

            @plsc.parallel_loop(0, C)
            def row_body(r):
                for j in range(D // _LANES):
                    sl = pl.ds(j * _LANES, _LANES)
                    plsc.addupdate(g_v.at[bg, r, sl], x_v[bx, r, sl])

            pltpu.async_copy(g_v.at[bg],
                             out_hbm.at[pl.ds(base + c * C, C), :], osem[bg])

        issue_in(0, 0, 0)

        def group(i, carry):
            for k_ in range(4):
                step(4 * i + k_, k_)
            return carry

        lax.fori_loop(0, n_chunks // 4, group, 0)
        for c in (n_chunks - 3, n_chunks - 2, n_chunks - 1):
            wait_out(c % 4)

    return k


def kernel(x, pos_ids, weight):
    B, L, D = x.shape
    V = weight.shape[0]
    N = B * L
    x_flat = x.reshape(N, D)
    idx_flat = pos_ids.reshape(N).astype(jnp.int32)
    out = _build(N, D, V)(x_flat, idx_flat, weight)
    return out.reshape(B, L, D)
